# Initial kernel scaffold; baseline (speedup 1.0000x reference)
import jax
import jax.numpy as jnp
from jax.experimental import pallas as pl


def kernel(edge_index, x, edge_attr, We1, be1, We2, be2, We3, be3, Wn1, bn1, Wn2, bn2, Wn3, bn3):
    raise NotImplementedError("write your pallas kernel here")


# speedup vs baseline: 4.1548x; 4.1548x over previous
import functools

import jax
import jax.numpy as jnp
from jax import lax
from jax.experimental import pallas as pl
from jax.experimental.pallas import tpu as pltpu
from jax.experimental.pallas import tpu_sc as plsc

N = 10000
E = 320000
DF = 128
DE = 16
HE = 128
HN = 128

NC = 2
NS = 16
NW = NC * NS
EPW = E // NW
CB = 80
NCH = EPW // CB
NP = 10240
RPT = NP // NS

EB = 2560
NB = 2000


def _softplus(x):
    return jnp.maximum(x, 0.0) + jnp.log1p(jnp.exp(-jnp.abs(x)))


def _precomp_body(x_ref, wa_ref, wb_ref, p1_ref, p2_ref):
    xb = x_ref[...]
    p1_ref[...] = jnp.dot(xb, wa_ref[...], preferred_element_type=jnp.float32)
    p2_ref[...] = jnp.dot(xb, wb_ref[...], preferred_element_type=jnp.float32)


def _edge_mlp_body(g1_ref, g2_ref, ea_ref, w1c_ref, b1_ref, w2_ref, b2_ref,
                   w3_ref, b3_ref, he_ref):
    h = (g1_ref[...] + g2_ref[...]
         + jnp.dot(ea_ref[...], w1c_ref[...], preferred_element_type=jnp.float32)
         + b1_ref[...])
    h = _softplus(h)
    h = _softplus(jnp.dot(h, w2_ref[...], preferred_element_type=jnp.float32)
                  + b2_ref[...])
    he_ref[...] = (jnp.dot(h, w3_ref[...], preferred_element_type=jnp.float32)
                   + b3_ref[...])


def _node_mlp_body(x_ref, agg_ref, wna_ref, wnb_ref, b1_ref, w2_ref, b2_ref,
                   w3_ref, b3_ref, hn_ref):
    a = agg_ref[0] + agg_ref[1]
    g = (jnp.dot(x_ref[...], wna_ref[...], preferred_element_type=jnp.float32)
         + jnp.dot(a, wnb_ref[...], preferred_element_type=jnp.float32)
         + b1_ref[...])
    g = _softplus(g)
    g = _softplus(jnp.dot(g, w2_ref[...], preferred_element_type=jnp.float32)
                  + b2_ref[...])
    hn_ref[...] = (jnp.dot(g, w3_ref[...], preferred_element_type=jnp.float32)
                   + b3_ref[...])


def _gather_body(p1, p2, srcr, dstr, g1, g2,
                 idx_s, idx_d, b1a, b1b, b2a, b2b, s1a, s1b, s2a, s2b):
    cid = lax.axis_index("c")
    sid = lax.axis_index("s")
    wid = sid * NC + cid
    base = wid * EPW
    pltpu.sync_copy(srcr.at[wid], idx_s)
    pltpu.sync_copy(dstr.at[wid], idx_d)
    bufs1 = (b1a, b1b)
    bufs2 = (b2a, b2b)
    sems1 = (s1a, s1b)
    sems2 = (s2a, s2b)
    for b in range(2):
        pltpu.async_copy(p1.at[idx_s.at[b]], bufs1[b], sems1[b])
        pltpu.async_copy(p2.at[idx_d.at[b]], bufs2[b], sems2[b])

    def step(t, carry):
        j0 = 2 * t
        for b in range(2):
            j = j0 + b
            pltpu.make_async_copy(p1.at[idx_s.at[j]], bufs1[b], sems1[b]).wait()
            pltpu.sync_copy(bufs1[b], g1.at[pl.ds(base + j * CB, CB)])
            pltpu.make_async_copy(p2.at[idx_d.at[j]], bufs2[b], sems2[b]).wait()
            pltpu.sync_copy(bufs2[b], g2.at[pl.ds(base + j * CB, CB)])

            @pl.when(j + 2 < NCH)
            def _():
                pltpu.async_copy(p1.at[idx_s.at[j + 2]], bufs1[b], sems1[b])
                pltpu.async_copy(p2.at[idx_d.at[j + 2]], bufs2[b], sems2[b])
        return carry

    lax.fori_loop(0, (NCH - 1) // 2, step, 0)
    j = NCH - 1
    pltpu.make_async_copy(p1.at[idx_s.at[j]], b1a, s1a).wait()
    pltpu.sync_copy(b1a, g1.at[pl.ds(base + j * CB, CB)])
    pltpu.make_async_copy(p2.at[idx_d.at[j]], b2a, s2a).wait()
    pltpu.sync_copy(b2a, g2.at[pl.ds(base + j * CB, CB)])


def _scatter_body(he, dstr, zz, aggp, idx_d, ba, bb, sa, sb, acc):
    cid = lax.axis_index("c")
    sid = lax.axis_index("s")
    wid = sid * NC + cid
    base = wid * EPW
    pltpu.sync_copy(dstr.at[wid], idx_d)
    pltpu.sync_copy(zz.at[pl.ds(sid * RPT, RPT)], acc.at[pl.ds(sid * RPT, RPT)])
    plsc.subcore_barrier()
    bufs = (ba, bb)
    sems = (sa, sb)
    for b in range(2):
        pltpu.async_copy(he.at[pl.ds(base + b * CB, CB)], bufs[b], sems[b])

    def step(t, carry):
        j0 = 2 * t
        for b in range(2):
            j = j0 + b
            pltpu.make_async_copy(he.at[pl.ds(base + j * CB, CB)],
                                  bufs[b], sems[b]).wait()
            pltpu.sync_copy(bufs[b], acc.at[idx_d.at[j]], add=True)

            @pl.when(j + 2 < NCH)
            def _():
                pltpu.async_copy(he.at[pl.ds(base + (j + 2) * CB, CB)],
                                 bufs[b], sems[b])
        return carry

    lax.fori_loop(0, (NCH - 1) // 2, step, 0)
    j = NCH - 1
    pltpu.make_async_copy(he.at[pl.ds(base + j * CB, CB)], ba, sa).wait()
    pltpu.sync_copy(ba, acc.at[idx_d.at[j]], add=True)
    plsc.subcore_barrier()
    pltpu.sync_copy(acc.at[pl.ds(sid * RPT, RPT)],
                    aggp.at[cid, pl.ds(sid * RPT, RPT)])


_SC_MESH = plsc.VectorSubcoreMesh(core_axis_name="c", subcore_axis_name="s")

_gather_call = functools.partial(
    pl.kernel,
    mesh=_SC_MESH,
    out_type=(jax.ShapeDtypeStruct((E, HE), jnp.float32),
              jax.ShapeDtypeStruct((E, HE), jnp.float32)),
    scratch_types=[
        pltpu.VMEM((NCH, CB), jnp.int32),
        pltpu.VMEM((NCH, CB), jnp.int32),
        pltpu.VMEM((CB, HE), jnp.float32),
        pltpu.VMEM((CB, HE), jnp.float32),
        pltpu.VMEM((CB, HE), jnp.float32),
        pltpu.VMEM((CB, HE), jnp.float32),
        pltpu.SemaphoreType.DMA,
        pltpu.SemaphoreType.DMA,
        pltpu.SemaphoreType.DMA,
        pltpu.SemaphoreType.DMA,
    ],
)(_gather_body)

_scatter_call = functools.partial(
    pl.kernel,
    mesh=_SC_MESH,
    out_type=jax.ShapeDtypeStruct((NC, NP, HE), jnp.float32),
    scratch_types=[
        pltpu.VMEM((NCH, CB), jnp.int32),
        pltpu.VMEM((CB, HE), jnp.float32),
        pltpu.VMEM((CB, HE), jnp.float32),
        pltpu.SemaphoreType.DMA,
        pltpu.SemaphoreType.DMA,
        pltpu.VMEM_SHARED((NP, HE), jnp.float32),
    ],
)(_scatter_body)


def kernel(edge_index, x, edge_attr, We1, be1, We2, be2, We3, be3,
           Wn1, bn1, Wn2, bn2, Wn3, bn3):
    We1a = We1[:DF]
    We1b = We1[DF:2 * DF]
    We1c = We1[2 * DF:]
    Wn1a = Wn1[:DF]
    Wn1b = Wn1[DF:]
    srcr = edge_index[0].reshape(NW, NCH, CB)
    dstr = edge_index[1].reshape(NW, NCH, CB)
    zeros = jnp.zeros((NP, HE), jnp.float32)
    be1r = be1.reshape(1, HE)
    be2r = be2.reshape(1, HE)
    be3r = be3.reshape(1, HE)
    bn1r = bn1.reshape(1, HN)
    bn2r = bn2.reshape(1, HN)
    bn3r = bn3.reshape(1, HN)

    p1, p2 = pl.pallas_call(
        _precomp_body,
        grid=(N // NB,),
        in_specs=[
            pl.BlockSpec((NB, DF), lambda i: (i, 0)),
            pl.BlockSpec((DF, HE), lambda i: (0, 0)),
            pl.BlockSpec((DF, HE), lambda i: (0, 0)),
        ],
        out_specs=[
            pl.BlockSpec((NB, HE), lambda i: (i, 0)),
            pl.BlockSpec((NB, HE), lambda i: (i, 0)),
        ],
        out_shape=[
            jax.ShapeDtypeStruct((N, HE), jnp.float32),
            jax.ShapeDtypeStruct((N, HE), jnp.float32),
        ],
    )(x, We1a, We1b)

    g1, g2 = _gather_call(p1, p2, srcr, dstr)

    h_e = pl.pallas_call(
        _edge_mlp_body,
        grid=(E // EB,),
        in_specs=[
            pl.BlockSpec((EB, HE), lambda i: (i, 0)),
            pl.BlockSpec((EB, HE), lambda i: (i, 0)),
            pl.BlockSpec((EB, DE), lambda i: (i, 0)),
            pl.BlockSpec((DE, HE), lambda i: (0, 0)),
            pl.BlockSpec((1, HE), lambda i: (0, 0)),
            pl.BlockSpec((HE, HE), lambda i: (0, 0)),
            pl.BlockSpec((1, HE), lambda i: (0, 0)),
            pl.BlockSpec((HE, HE), lambda i: (0, 0)),
            pl.BlockSpec((1, HE), lambda i: (0, 0)),
        ],
        out_specs=pl.BlockSpec((EB, HE), lambda i: (i, 0)),
        out_shape=jax.ShapeDtypeStruct((E, HE), jnp.float32),
    )(g1, g2, edge_attr, We1c, be1r, We2, be2r, We3, be3r)

    aggp = _scatter_call(h_e, dstr, zeros)

    h_n = pl.pallas_call(
        _node_mlp_body,
        grid=(N // NB,),
        in_specs=[
            pl.BlockSpec((NB, DF), lambda i: (i, 0)),
            pl.BlockSpec((NC, NB, HE), lambda i: (0, i, 0)),
            pl.BlockSpec((DF, HN), lambda i: (0, 0)),
            pl.BlockSpec((HE, HN), lambda i: (0, 0)),
            pl.BlockSpec((1, HN), lambda i: (0, 0)),
            pl.BlockSpec((HN, HN), lambda i: (0, 0)),
            pl.BlockSpec((1, HN), lambda i: (0, 0)),
            pl.BlockSpec((HN, HN), lambda i: (0, 0)),
            pl.BlockSpec((1, HN), lambda i: (0, 0)),
        ],
        out_specs=pl.BlockSpec((NB, HN), lambda i: (i, 0)),
        out_shape=jax.ShapeDtypeStruct((N, HN), jnp.float32),
    )(x, aggp, Wn1a, Wn1b, bn1r, Wn2, bn2r, Wn3, bn3r)

    return (h_e, h_n)


# speedup vs baseline: 4.2932x; 1.0333x over previous
import functools

import jax
import jax.numpy as jnp
from jax import lax
from jax.experimental import pallas as pl
from jax.experimental.pallas import tpu as pltpu
from jax.experimental.pallas import tpu_sc as plsc

N = 10000
E = 320000
DF = 128
DE = 16
HE = 128
HN = 128

NC = 2
NS = 16
NW = NC * NS
EPW = E // NW
CB = 80
NCH = EPW // CB
NP = 10240
RPT = NP // NS

EB = 2560
NB = 2000


_LOG2E = 1.4426950408889634
_LN2 = 0.6931471805599453


def _softplus(x):
    p = jnp.exp2(jnp.abs(x) * -_LOG2E)
    return jnp.maximum(x, 0.0) + jnp.log2(1.0 + p) * _LN2


def _precomp_body(x_ref, wa_ref, wb_ref, p1_ref, p2_ref):
    xb = x_ref[...]
    p1_ref[...] = jnp.dot(xb, wa_ref[...], preferred_element_type=jnp.float32)
    p2_ref[...] = jnp.dot(xb, wb_ref[...], preferred_element_type=jnp.float32)


def _edge_mlp_body(g1_ref, g2_ref, ea_ref, w1c_ref, b1_ref, w2_ref, b2_ref,
                   w3_ref, b3_ref, he_ref):
    h = (g1_ref[...] + g2_ref[...]
         + jnp.dot(ea_ref[...], w1c_ref[...], preferred_element_type=jnp.float32)
         + b1_ref[...])
    h = _softplus(h)
    h = _softplus(jnp.dot(h, w2_ref[...], preferred_element_type=jnp.float32)
                  + b2_ref[...])
    he_ref[...] = (jnp.dot(h, w3_ref[...], preferred_element_type=jnp.float32)
                   + b3_ref[...])


def _node_mlp_body(x_ref, agg_ref, wna_ref, wnb_ref, b1_ref, w2_ref, b2_ref,
                   w3_ref, b3_ref, hn_ref):
    a = agg_ref[0] + agg_ref[1]
    g = (jnp.dot(x_ref[...], wna_ref[...], preferred_element_type=jnp.float32)
         + jnp.dot(a, wnb_ref[...], preferred_element_type=jnp.float32)
         + b1_ref[...])
    g = _softplus(g)
    g = _softplus(jnp.dot(g, w2_ref[...], preferred_element_type=jnp.float32)
                  + b2_ref[...])
    hn_ref[...] = (jnp.dot(g, w3_ref[...], preferred_element_type=jnp.float32)
                   + b3_ref[...])


def _gather_body(p1, p2, srcr, dstr, g1, g2,
                 idx_s, idx_d, b1a, b1b, b2a, b2b, s1a, s1b, s2a, s2b):
    cid = lax.axis_index("c")
    sid = lax.axis_index("s")
    wid = sid * NC + cid
    base = wid * EPW
    pltpu.sync_copy(srcr.at[wid], idx_s)
    pltpu.sync_copy(dstr.at[wid], idx_d)
    bufs1 = (b1a, b1b)
    bufs2 = (b2a, b2b)
    sems1 = (s1a, s1b)
    sems2 = (s2a, s2b)
    for b in range(2):
        pltpu.async_copy(p1.at[idx_s.at[b]], bufs1[b], sems1[b])
        pltpu.async_copy(p2.at[idx_d.at[b]], bufs2[b], sems2[b])

    def step(t, carry):
        j0 = 2 * t
        for b in range(2):
            j = j0 + b
            pltpu.make_async_copy(p1.at[idx_s.at[j]], bufs1[b], sems1[b]).wait()
            pltpu.sync_copy(bufs1[b], g1.at[pl.ds(base + j * CB, CB)])
            pltpu.make_async_copy(p2.at[idx_d.at[j]], bufs2[b], sems2[b]).wait()
            pltpu.sync_copy(bufs2[b], g2.at[pl.ds(base + j * CB, CB)])

            @pl.when(j + 2 < NCH)
            def _():
                pltpu.async_copy(p1.at[idx_s.at[j + 2]], bufs1[b], sems1[b])
                pltpu.async_copy(p2.at[idx_d.at[j + 2]], bufs2[b], sems2[b])
        return carry

    lax.fori_loop(0, (NCH - 1) // 2, step, 0)
    j = NCH - 1
    pltpu.make_async_copy(p1.at[idx_s.at[j]], b1a, s1a).wait()
    pltpu.sync_copy(b1a, g1.at[pl.ds(base + j * CB, CB)])
    pltpu.make_async_copy(p2.at[idx_d.at[j]], b2a, s2a).wait()
    pltpu.sync_copy(b2a, g2.at[pl.ds(base + j * CB, CB)])


def _scatter_body(he, dstr, zz, aggp, idx_d, ba, bb, sa, sb, acc):
    cid = lax.axis_index("c")
    sid = lax.axis_index("s")
    wid = sid * NC + cid
    base = wid * EPW
    pltpu.sync_copy(dstr.at[wid], idx_d)
    pltpu.sync_copy(zz.at[pl.ds(sid * RPT, RPT)], acc.at[pl.ds(sid * RPT, RPT)])
    plsc.subcore_barrier()
    bufs = (ba, bb)
    sems = (sa, sb)
    for b in range(2):
        pltpu.async_copy(he.at[pl.ds(base + b * CB, CB)], bufs[b], sems[b])

    def step(t, carry):
        j0 = 2 * t
        for b in range(2):
            j = j0 + b
            pltpu.make_async_copy(he.at[pl.ds(base + j * CB, CB)],
                                  bufs[b], sems[b]).wait()
            pltpu.sync_copy(bufs[b], acc.at[idx_d.at[j]], add=True)

            @pl.when(j + 2 < NCH)
            def _():
                pltpu.async_copy(he.at[pl.ds(base + (j + 2) * CB, CB)],
                                 bufs[b], sems[b])
        return carry

    lax.fori_loop(0, (NCH - 1) // 2, step, 0)
    j = NCH - 1
    pltpu.make_async_copy(he.at[pl.ds(base + j * CB, CB)], ba, sa).wait()
    pltpu.sync_copy(ba, acc.at[idx_d.at[j]], add=True)
    plsc.subcore_barrier()
    pltpu.sync_copy(acc.at[pl.ds(sid * RPT, RPT)],
                    aggp.at[cid, pl.ds(sid * RPT, RPT)])


_SC_MESH = plsc.VectorSubcoreMesh(core_axis_name="c", subcore_axis_name="s")

_gather_call = functools.partial(
    pl.kernel,
    mesh=_SC_MESH,
    out_type=(jax.ShapeDtypeStruct((E, HE), jnp.float32),
              jax.ShapeDtypeStruct((E, HE), jnp.float32)),
    scratch_types=[
        pltpu.VMEM((NCH, CB), jnp.int32),
        pltpu.VMEM((NCH, CB), jnp.int32),
        pltpu.VMEM((CB, HE), jnp.float32),
        pltpu.VMEM((CB, HE), jnp.float32),
        pltpu.VMEM((CB, HE), jnp.float32),
        pltpu.VMEM((CB, HE), jnp.float32),
        pltpu.SemaphoreType.DMA,
        pltpu.SemaphoreType.DMA,
        pltpu.SemaphoreType.DMA,
        pltpu.SemaphoreType.DMA,
    ],
)(_gather_body)

_scatter_call = functools.partial(
    pl.kernel,
    mesh=_SC_MESH,
    out_type=jax.ShapeDtypeStruct((NC, NP, HE), jnp.float32),
    scratch_types=[
        pltpu.VMEM((NCH, CB), jnp.int32),
        pltpu.VMEM((CB, HE), jnp.float32),
        pltpu.VMEM((CB, HE), jnp.float32),
        pltpu.SemaphoreType.DMA,
        pltpu.SemaphoreType.DMA,
        pltpu.VMEM_SHARED((NP, HE), jnp.float32),
    ],
)(_scatter_body)


def kernel(edge_index, x, edge_attr, We1, be1, We2, be2, We3, be3,
           Wn1, bn1, Wn2, bn2, Wn3, bn3):
    We1a = We1[:DF]
    We1b = We1[DF:2 * DF]
    We1c = We1[2 * DF:]
    Wn1a = Wn1[:DF]
    Wn1b = Wn1[DF:]
    srcr = edge_index[0].reshape(NW, NCH, CB)
    dstr = edge_index[1].reshape(NW, NCH, CB)
    zeros = jnp.zeros((NP, HE), jnp.float32)
    be1r = be1.reshape(1, HE)
    be2r = be2.reshape(1, HE)
    be3r = be3.reshape(1, HE)
    bn1r = bn1.reshape(1, HN)
    bn2r = bn2.reshape(1, HN)
    bn3r = bn3.reshape(1, HN)

    p1, p2 = pl.pallas_call(
        _precomp_body,
        grid=(N // NB,),
        in_specs=[
            pl.BlockSpec((NB, DF), lambda i: (i, 0)),
            pl.BlockSpec((DF, HE), lambda i: (0, 0)),
            pl.BlockSpec((DF, HE), lambda i: (0, 0)),
        ],
        out_specs=[
            pl.BlockSpec((NB, HE), lambda i: (i, 0)),
            pl.BlockSpec((NB, HE), lambda i: (i, 0)),
        ],
        out_shape=[
            jax.ShapeDtypeStruct((N, HE), jnp.float32),
            jax.ShapeDtypeStruct((N, HE), jnp.float32),
        ],
    )(x, We1a, We1b)

    g1, g2 = _gather_call(p1, p2, srcr, dstr)

    h_e = pl.pallas_call(
        _edge_mlp_body,
        grid=(E // EB,),
        in_specs=[
            pl.BlockSpec((EB, HE), lambda i: (i, 0)),
            pl.BlockSpec((EB, HE), lambda i: (i, 0)),
            pl.BlockSpec((EB, DE), lambda i: (i, 0)),
            pl.BlockSpec((DE, HE), lambda i: (0, 0)),
            pl.BlockSpec((1, HE), lambda i: (0, 0)),
            pl.BlockSpec((HE, HE), lambda i: (0, 0)),
            pl.BlockSpec((1, HE), lambda i: (0, 0)),
            pl.BlockSpec((HE, HE), lambda i: (0, 0)),
            pl.BlockSpec((1, HE), lambda i: (0, 0)),
        ],
        out_specs=pl.BlockSpec((EB, HE), lambda i: (i, 0)),
        out_shape=jax.ShapeDtypeStruct((E, HE), jnp.float32),
    )(g1, g2, edge_attr, We1c, be1r, We2, be2r, We3, be3r)

    aggp = _scatter_call(h_e, dstr, zeros)

    h_n = pl.pallas_call(
        _node_mlp_body,
        grid=(N // NB,),
        in_specs=[
            pl.BlockSpec((NB, DF), lambda i: (i, 0)),
            pl.BlockSpec((NC, NB, HE), lambda i: (0, i, 0)),
            pl.BlockSpec((DF, HN), lambda i: (0, 0)),
            pl.BlockSpec((HE, HN), lambda i: (0, 0)),
            pl.BlockSpec((1, HN), lambda i: (0, 0)),
            pl.BlockSpec((HN, HN), lambda i: (0, 0)),
            pl.BlockSpec((1, HN), lambda i: (0, 0)),
            pl.BlockSpec((HN, HN), lambda i: (0, 0)),
            pl.BlockSpec((1, HN), lambda i: (0, 0)),
        ],
        out_specs=pl.BlockSpec((NB, HN), lambda i: (i, 0)),
        out_shape=jax.ShapeDtypeStruct((N, HN), jnp.float32),
    )(x, aggp, Wn1a, Wn1b, bn1r, Wn2, bn2r, Wn3, bn3r)

    return (h_e, h_n)
